# baseline (speedup 1.0000x reference)
import functools

import jax
import jax.numpy as jnp
from jax import lax
from jax.experimental import pallas as pl
from jax.experimental.pallas import tpu as pltpu
from jax.experimental.pallas import tpu_sc as plsc

N = 10000
E = 320000
D = 128

NC_SC = 2
NS = 16
NW = NC_SC * NS
CH = 128
NCHUNK = 80
EPT = CH * NCHUNK
EPAD = EPT * NW
N_PAD = 10240
RPT = N_PAD // NS
RING = 2
DRING = 4
DEGW = 16

_mesh = plsc.VectorSubcoreMesh(core_axis_name="c", subcore_axis_name="s")


def _zero_buf(buf, nrows, ncols):
    @pl.loop(0, nrows)
    def _(r):
        @pl.loop(0, ncols, step=16)
        def _(cc):
            buf[r, pl.ds(cc, 16)] = jnp.zeros((16,), jnp.float32)


def _fill_ones(buf, nrows):
    @pl.loop(0, nrows)
    def _(r):
        buf[r, pl.ds(0, 16)] = jnp.ones((16,), jnp.float32)


@functools.partial(
    pl.kernel,
    out_type=jax.ShapeDtypeStruct((NC_SC, N_PAD, DEGW), jnp.float32),
    mesh=_mesh,
    scratch_types=[
        pltpu.VMEM_SHARED((N_PAD, DEGW), jnp.float32),
        pltpu.VMEM((NCHUNK, CH), jnp.int32),
        pltpu.VMEM((CH, DEGW), jnp.float32),
        pltpu.SemaphoreType.DMA,
        pltpu.SemaphoreType.DMA,
        pltpu.SemaphoreType.DMA,
        pltpu.SemaphoreType.DMA,
    ],
)
def _sc_deg(dst_hbm, out_hbm, deg_acc, dst_v, ones_v, s0, s1, s2, s3):
    ssem = (s0, s1, s2, s3)
    c = lax.axis_index("c")
    s = lax.axis_index("s")
    wid = c * NS + s

    _zero_buf(ones_v, CH, DEGW)
    for k in range(RPT // CH):
        pltpu.sync_copy(ones_v,
                        deg_acc.at[pl.ds(s * RPT + k * CH, CH)])
    _fill_ones(ones_v, CH)
    pltpu.sync_copy(dst_hbm.at[pl.ds(wid * NCHUNK, NCHUNK)], dst_v)
    plsc.subcore_barrier()

    @pl.loop(0, NCHUNK, step=DRING)
    def _(g0):
        for b in range(DRING):
            i = g0 + b

            @pl.when(i >= DRING)
            def _():
                pltpu.make_async_copy(ones_v, deg_acc.at[dst_v.at[i]],
                                      ssem[b]).wait()

            pltpu.async_copy(ones_v, deg_acc.at[dst_v.at[i]], ssem[b],
                             add=True)

    for b in range(DRING):
        pltpu.make_async_copy(ones_v, deg_acc.at[dst_v.at[0]], ssem[b]).wait()
    plsc.subcore_barrier()
    pltpu.sync_copy(deg_acc.at[pl.ds(s * RPT, RPT)],
                    out_hbm.at[c].at[pl.ds(s * RPT, RPT)])


MCH = 128
MNCHUNK = 84
MEPT = MCH * MNCHUNK
MEPAD = MEPT * NW
MR = 3
MG = 2
MS = MR - MG
MIR = 6
MRPT = N // NS


@functools.partial(
    pl.kernel,
    out_type=jax.ShapeDtypeStruct((NC_SC, N, D), jnp.float32),
    mesh=_mesh,
    scratch_types=(
        [pltpu.VMEM_SHARED((N, D), jnp.float32)]
        + [pltpu.VMEM((1, 2, MCH), jnp.int32)] * MIR
        + [pltpu.VMEM((MCH, D), jnp.float32)] * MR
        + [pltpu.SemaphoreType.DMA] * (MIR + 2 * MR)
    ),
)
def _sc_msg(s_hbm, gidx_hbm, out_hbm, acc, *refs):
    ix = refs[:MIR]
    rows = refs[MIR:MIR + MR]
    isem = refs[MIR + MR:2 * MIR + MR]
    gsem = refs[2 * MIR + MR:2 * MIR + 2 * MR]
    ssem = refs[2 * MIR + 2 * MR:]
    c = lax.axis_index("c")
    s = lax.axis_index("s")
    wid = c * NS + s
    base = wid * MNCHUNK

    def idx_load(chunk, slot):
        pltpu.async_copy(gidx_hbm.at[pl.ds(base + chunk, 1)], ix[slot],
                         isem[slot])

    def idx_wait(slot):
        pltpu.make_async_copy(gidx_hbm.at[pl.ds(base, 1)], ix[slot],
                              isem[slot]).wait()

    def gather(bslot, islot):
        pltpu.async_copy(s_hbm.at[ix[islot].at[0, 0]], rows[bslot],
                         gsem[bslot])

    def gather_wait(bslot):
        pltpu.make_async_copy(s_hbm.at[ix[0].at[0, 0]], rows[bslot],
                              gsem[bslot]).wait()

    def scatter(bslot, islot):
        pltpu.async_copy(rows[bslot], acc.at[ix[islot].at[0, 1]],
                         ssem[bslot], add=True)

    def scatter_wait(bslot):
        pltpu.make_async_copy(rows[bslot], acc.at[ix[0].at[0, 1]],
                              ssem[bslot]).wait()

    _zero_buf(rows[0], MCH, D)
    for j in range(5):
        ck = s + 16 * j

        @pl.when(ck < N // MCH)
        def _():
            pltpu.sync_copy(rows[0], acc.at[pl.ds(ck * MCH, MCH)])

        @pl.when(ck == N // MCH)
        def _():
            pltpu.sync_copy(rows[0].at[pl.ds(0, N % MCH)],
                            acc.at[pl.ds(ck * MCH, N % MCH)])

    for b in range(MIR):
        idx_load(b, b)
    for b in range(MG):
        idx_wait(b)
        gather(b, b)
    plsc.subcore_barrier()

    @pl.loop(0, MNCHUNK, step=MIR)
    def _(g0):
        for u in range(MIR):
            i = g0 + u
            b = u % MR
            gather_wait(b)
            scatter(b, u)
            scatter_wait(b)
            ni = i + MG
            bn = (u + MG) % MR
            un = (u + MG) % MIR

            @pl.when(ni < MNCHUNK)
            def _():
                idx_wait(un)
                gather(bn, un)

            nl = i + MIR

            @pl.when(nl < MNCHUNK)
            def _():
                idx_load(nl, u)

    plsc.subcore_barrier()
    for j in range(5):
        ck = s + 16 * j

        @pl.when(ck < N // MCH)
        def _():
            pltpu.sync_copy(acc.at[pl.ds(ck * MCH, MCH)],
                            out_hbm.at[c].at[pl.ds(ck * MCH, MCH)])

        @pl.when(ck == N // MCH)
        def _():
            pltpu.sync_copy(acc.at[pl.ds(ck * MCH, N % MCH)],
                            out_hbm.at[c].at[pl.ds(ck * MCH, N % MCH)])


_RB = 1000
_GRID = N // _RB


def _mm_body(x_ref, w_ref, o_ref):
    o_ref[...] = jnp.dot(x_ref[...], w_ref[...],
                         preferred_element_type=jnp.float32)


def _tc_matmul(x, w):
    return pl.pallas_call(
        _mm_body,
        grid=(_GRID,),
        in_specs=[
            pl.BlockSpec((_RB, D), lambda i: (i, 0)),
            pl.BlockSpec((D, D), lambda i: (0, 0)),
        ],
        out_specs=pl.BlockSpec((_RB, D), lambda i: (i, 0)),
        out_shape=jax.ShapeDtypeStruct((N, D), jnp.float32),
    )(x, w)


def _scale_body(d0_ref, d1_ref, hw_ref, s_ref, dn_ref):
    deg = d0_ref[:, :1] + d1_ref[:, :1] + 1.0
    dn = lax.rsqrt(deg)
    dn_ref[...] = dn
    s_ref[...] = hw_ref[...] * dn


def _tc_scale(d0, d1, hw):
    return pl.pallas_call(
        _scale_body,
        grid=(_GRID,),
        in_specs=[
            pl.BlockSpec((_RB, DEGW), lambda i: (i, 0)),
            pl.BlockSpec((_RB, DEGW), lambda i: (i, 0)),
            pl.BlockSpec((_RB, D), lambda i: (i, 0)),
        ],
        out_specs=[
            pl.BlockSpec((_RB, D), lambda i: (i, 0)),
            pl.BlockSpec((_RB, 1), lambda i: (i, 0)),
        ],
        out_shape=[
            jax.ShapeDtypeStruct((N, D), jnp.float32),
            jax.ShapeDtypeStruct((N, 1), jnp.float32),
        ],
    )(d0, d1, hw)


def _layer_body(a0_ref, a1_ref, sp_ref, dn_ref, b_ref, w_ref, o_ref):
    dn = dn_ref[...]
    t = (a0_ref[...] + a1_ref[...] + sp_ref[...]) * dn + b_ref[...]
    h = jnp.maximum(t, 0.0)
    o_ref[...] = jnp.dot(h, w_ref[...],
                         preferred_element_type=jnp.float32) * dn


def _tc_layer(a0, a1, sp, dn, bias, w):
    return pl.pallas_call(
        _layer_body,
        grid=(_GRID,),
        in_specs=[
            pl.BlockSpec((_RB, D), lambda i: (i, 0)),
            pl.BlockSpec((_RB, D), lambda i: (i, 0)),
            pl.BlockSpec((_RB, D), lambda i: (i, 0)),
            pl.BlockSpec((_RB, 1), lambda i: (i, 0)),
            pl.BlockSpec((1, D), lambda i: (0, 0)),
            pl.BlockSpec((D, D), lambda i: (0, 0)),
        ],
        out_specs=pl.BlockSpec((_RB, D), lambda i: (i, 0)),
        out_shape=jax.ShapeDtypeStruct((N, D), jnp.float32),
    )(a0, a1, sp, dn, bias, w)


def _final_body(a0_ref, a1_ref, sp_ref, dn_ref, b_ref, o_ref):
    t = (a0_ref[...] + a1_ref[...] + sp_ref[...]) * dn_ref[...] + b_ref[...]
    o_ref[...] = jnp.maximum(t, 0.0)


def _tc_final(a0, a1, sp, dn, bias):
    return pl.pallas_call(
        _final_body,
        grid=(_GRID,),
        in_specs=[
            pl.BlockSpec((_RB, D), lambda i: (i, 0)),
            pl.BlockSpec((_RB, D), lambda i: (i, 0)),
            pl.BlockSpec((_RB, D), lambda i: (i, 0)),
            pl.BlockSpec((_RB, 1), lambda i: (i, 0)),
            pl.BlockSpec((1, D), lambda i: (0, 0)),
        ],
        out_specs=pl.BlockSpec((_RB, D), lambda i: (i, 0)),
        out_shape=jax.ShapeDtypeStruct((N, D), jnp.float32),
    )(a0, a1, sp, dn, bias)


_ZPAD = 16


def _pad_s(sv):
    return jnp.concatenate([sv, jnp.zeros((_ZPAD, D), jnp.float32)])


def kernel(x, g, W0, b0, W1, b1, W2, b2):
    dpad = EPAD - E
    dstp = jnp.concatenate([g[1], jnp.full((dpad,), N, jnp.int32)])
    dstp = dstp.reshape(EPAD // CH, CH)

    mpad = MEPAD - E
    msrc = jnp.concatenate([g[0], jnp.full((mpad,), N, jnp.int32)])
    mdst = jnp.concatenate([g[1], jnp.zeros((mpad,), jnp.int32)])
    gidx = jnp.stack([msrc.reshape(MEPAD // MCH, MCH),
                      mdst.reshape(MEPAD // MCH, MCH)], axis=1)

    degp = _sc_deg(dstp)
    hw0 = _tc_matmul(x, W0)
    s0, dn = _tc_scale(degp[0, :N], degp[1, :N], hw0)

    acc = _sc_msg(_pad_s(s0), gidx)
    s1 = _tc_layer(acc[0], acc[1], s0, dn, b0.reshape(1, D), W1)
    acc = _sc_msg(_pad_s(s1), gidx)
    s2 = _tc_layer(acc[0], acc[1], s1, dn, b1.reshape(1, D), W2)
    acc = _sc_msg(_pad_s(s2), gidx)
    return _tc_final(acc[0], acc[1], s2, dn, b2.reshape(1, D))


# speedup vs baseline: 6.1035x; 6.1035x over previous
import functools

import jax
import jax.numpy as jnp
import numpy as np
from jax import lax
from jax.experimental import pallas as pl
from jax.experimental.pallas import tpu as pltpu
from jax.experimental.pallas import tpu_sc as plsc

N = 10000
E = 320000
D = 128

NC_SC = 2
NS = 16
NW = NC_SC * NS
CH = 128
NCHUNK = 80
EPT = CH * NCHUNK
EPAD = EPT * NW
N_PAD = 10240
RPT = N_PAD // NS
RING = 2
DRING = 4
DEGW = 16

_mesh = plsc.VectorSubcoreMesh(core_axis_name="c", subcore_axis_name="s")


def _zero_buf(buf, nrows, ncols):
    @pl.loop(0, nrows)
    def _(r):
        @pl.loop(0, ncols, step=16)
        def _(cc):
            buf[r, pl.ds(cc, 16)] = jnp.zeros((16,), jnp.float32)


def _fill_ones(buf, nrows):
    @pl.loop(0, nrows)
    def _(r):
        buf[r, pl.ds(0, 16)] = jnp.ones((16,), jnp.float32)


@functools.partial(
    pl.kernel,
    out_type=jax.ShapeDtypeStruct((NC_SC, N_PAD, DEGW), jnp.float32),
    mesh=_mesh,
    scratch_types=[
        pltpu.VMEM_SHARED((N_PAD, DEGW), jnp.float32),
        pltpu.VMEM((NCHUNK, CH), jnp.int32),
        pltpu.VMEM((CH, DEGW), jnp.float32),
        pltpu.SemaphoreType.DMA,
        pltpu.SemaphoreType.DMA,
        pltpu.SemaphoreType.DMA,
        pltpu.SemaphoreType.DMA,
    ],
)
def _sc_deg(dst_hbm, out_hbm, deg_acc, dst_v, ones_v, s0, s1, s2, s3):
    ssem = (s0, s1, s2, s3)
    c = lax.axis_index("c")
    s = lax.axis_index("s")
    wid = c * NS + s

    _zero_buf(ones_v, CH, DEGW)
    for k in range(RPT // CH):
        pltpu.sync_copy(ones_v,
                        deg_acc.at[pl.ds(s * RPT + k * CH, CH)])
    _fill_ones(ones_v, CH)
    pltpu.sync_copy(dst_hbm.at[pl.ds(wid * NCHUNK, NCHUNK)], dst_v)
    plsc.subcore_barrier()

    @pl.loop(0, NCHUNK, step=DRING)
    def _(g0):
        for b in range(DRING):
            i = g0 + b

            @pl.when(i >= DRING)
            def _():
                pltpu.make_async_copy(ones_v, deg_acc.at[dst_v.at[i]],
                                      ssem[b]).wait()

            pltpu.async_copy(ones_v, deg_acc.at[dst_v.at[i]], ssem[b],
                             add=True)

    for b in range(DRING):
        pltpu.make_async_copy(ones_v, deg_acc.at[dst_v.at[0]], ssem[b]).wait()
    plsc.subcore_barrier()
    pltpu.sync_copy(deg_acc.at[pl.ds(s * RPT, RPT)],
                    out_hbm.at[c].at[pl.ds(s * RPT, RPT)])


MCH = 128
MNCHUNK = 84
MEPT = MCH * MNCHUNK
MEPAD = MEPT * NW
MR = 3
MG = 2
MS = MR - MG
MIR = 6
MRPT = N // NS


@functools.partial(
    pl.kernel,
    out_type=jax.ShapeDtypeStruct((NC_SC, N, D), jnp.float32),
    mesh=_mesh,
    scratch_types=(
        [pltpu.VMEM_SHARED((N, D), jnp.float32)]
        + [pltpu.VMEM((1, 2, MCH), jnp.int32)] * MIR
        + [pltpu.VMEM((MCH, D), jnp.float32)] * MR
        + [pltpu.SemaphoreType.DMA] * (MIR + 2 * MR)
    ),
)
def _sc_msg(s_hbm, gidx_hbm, out_hbm, acc, *refs):
    ix = refs[:MIR]
    rows = refs[MIR:MIR + MR]
    isem = refs[MIR + MR:2 * MIR + MR]
    gsem = refs[2 * MIR + MR:2 * MIR + 2 * MR]
    ssem = refs[2 * MIR + 2 * MR:]
    c = lax.axis_index("c")
    s = lax.axis_index("s")
    wid = c * NS + s
    base = wid * MNCHUNK

    def idx_load(chunk, slot):
        pltpu.async_copy(gidx_hbm.at[pl.ds(base + chunk, 1)], ix[slot],
                         isem[slot])

    def idx_wait(slot):
        pltpu.make_async_copy(gidx_hbm.at[pl.ds(base, 1)], ix[slot],
                              isem[slot]).wait()

    def gather(bslot, islot):
        pltpu.async_copy(s_hbm.at[ix[islot].at[0, 0]], rows[bslot],
                         gsem[bslot])

    def gather_wait(bslot):
        pltpu.make_async_copy(s_hbm.at[ix[0].at[0, 0]], rows[bslot],
                              gsem[bslot]).wait()

    def scatter(bslot, islot):
        pltpu.async_copy(rows[bslot], acc.at[ix[islot].at[0, 1]],
                         ssem[bslot], add=True)

    def scatter_wait(bslot):
        pltpu.make_async_copy(rows[bslot], acc.at[ix[0].at[0, 1]],
                              ssem[bslot]).wait()

    _zero_buf(rows[0], MCH, D)
    for j in range(5):
        ck = s + 16 * j

        @pl.when(ck < N // MCH)
        def _():
            pltpu.sync_copy(rows[0], acc.at[pl.ds(ck * MCH, MCH)])

        @pl.when(ck == N // MCH)
        def _():
            pltpu.sync_copy(rows[0].at[pl.ds(0, N % MCH)],
                            acc.at[pl.ds(ck * MCH, N % MCH)])

    for b in range(MIR):
        idx_load(b, b)
    for b in range(MG):
        idx_wait(b)
        gather(b, b)
    plsc.subcore_barrier()

    @pl.loop(0, MNCHUNK, step=MIR)
    def _(g0):
        for u in range(MIR):
            i = g0 + u
            b = u % MR
            gather_wait(b)
            scatter(b, u)
            scatter_wait(b)
            ni = i + MG
            bn = (u + MG) % MR
            un = (u + MG) % MIR

            @pl.when(ni < MNCHUNK)
            def _():
                idx_wait(un)
                gather(bn, un)

            nl = i + MIR

            @pl.when(nl < MNCHUNK)
            def _():
                idx_load(nl, u)

    plsc.subcore_barrier()
    for j in range(5):
        ck = s + 16 * j

        @pl.when(ck < N // MCH)
        def _():
            pltpu.sync_copy(acc.at[pl.ds(ck * MCH, MCH)],
                            out_hbm.at[c].at[pl.ds(ck * MCH, MCH)])

        @pl.when(ck == N // MCH)
        def _():
            pltpu.sync_copy(acc.at[pl.ds(ck * MCH, N % MCH)],
                            out_hbm.at[c].at[pl.ds(ck * MCH, N % MCH)])


_RB = 1000
_GRID = N // _RB


def _mm_body(x_ref, w_ref, o_ref):
    o_ref[...] = jnp.dot(x_ref[...], w_ref[...],
                         preferred_element_type=jnp.float32)


def _tc_matmul(x, w):
    return pl.pallas_call(
        _mm_body,
        grid=(_GRID,),
        in_specs=[
            pl.BlockSpec((_RB, D), lambda i: (i, 0)),
            pl.BlockSpec((D, D), lambda i: (0, 0)),
        ],
        out_specs=pl.BlockSpec((_RB, D), lambda i: (i, 0)),
        out_shape=jax.ShapeDtypeStruct((N, D), jnp.float32),
    )(x, w)


def _scale_body(d0_ref, d1_ref, hw_ref, s_ref, dn_ref):
    deg = d0_ref[:, :1] + d1_ref[:, :1] + 1.0
    dn = lax.rsqrt(deg)
    dn_ref[...] = dn
    s_ref[...] = hw_ref[...] * dn


def _tc_scale(d0, d1, hw):
    return pl.pallas_call(
        _scale_body,
        grid=(_GRID,),
        in_specs=[
            pl.BlockSpec((_RB, DEGW), lambda i: (i, 0)),
            pl.BlockSpec((_RB, DEGW), lambda i: (i, 0)),
            pl.BlockSpec((_RB, D), lambda i: (i, 0)),
        ],
        out_specs=[
            pl.BlockSpec((_RB, D), lambda i: (i, 0)),
            pl.BlockSpec((_RB, 1), lambda i: (i, 0)),
        ],
        out_shape=[
            jax.ShapeDtypeStruct((N, D), jnp.float32),
            jax.ShapeDtypeStruct((N, 1), jnp.float32),
        ],
    )(d0, d1, hw)


def _layer_body(a0_ref, a1_ref, sp_ref, dn_ref, b_ref, w_ref, o_ref):
    dn = dn_ref[...]
    t = (a0_ref[...] + a1_ref[...] + sp_ref[...]) * dn + b_ref[...]
    h = jnp.maximum(t, 0.0)
    o_ref[...] = jnp.dot(h, w_ref[...],
                         preferred_element_type=jnp.float32) * dn


def _tc_layer(a0, a1, sp, dn, bias, w):
    return pl.pallas_call(
        _layer_body,
        grid=(_GRID,),
        in_specs=[
            pl.BlockSpec((_RB, D), lambda i: (i, 0)),
            pl.BlockSpec((_RB, D), lambda i: (i, 0)),
            pl.BlockSpec((_RB, D), lambda i: (i, 0)),
            pl.BlockSpec((_RB, 1), lambda i: (i, 0)),
            pl.BlockSpec((1, D), lambda i: (0, 0)),
            pl.BlockSpec((D, D), lambda i: (0, 0)),
        ],
        out_specs=pl.BlockSpec((_RB, D), lambda i: (i, 0)),
        out_shape=jax.ShapeDtypeStruct((N, D), jnp.float32),
    )(a0, a1, sp, dn, bias, w)


def _final_body(a0_ref, a1_ref, sp_ref, dn_ref, b_ref, o_ref):
    t = (a0_ref[...] + a1_ref[...] + sp_ref[...]) * dn_ref[...] + b_ref[...]
    o_ref[...] = jnp.maximum(t, 0.0)


def _tc_final(a0, a1, sp, dn, bias):
    return pl.pallas_call(
        _final_body,
        grid=(_GRID,),
        in_specs=[
            pl.BlockSpec((_RB, D), lambda i: (i, 0)),
            pl.BlockSpec((_RB, D), lambda i: (i, 0)),
            pl.BlockSpec((_RB, D), lambda i: (i, 0)),
            pl.BlockSpec((_RB, 1), lambda i: (i, 0)),
            pl.BlockSpec((1, D), lambda i: (0, 0)),
        ],
        out_specs=pl.BlockSpec((_RB, D), lambda i: (i, 0)),
        out_shape=jax.ShapeDtypeStruct((N, D), jnp.float32),
    )(a0, a1, sp, dn, bias)


_ZPAD = 16


def _pad_s(sv):
    return jnp.concatenate([sv, jnp.zeros((_ZPAD, D), jnp.float32)])


def kernel(x, g, W0, b0, W1, b1, W2, b2):
    dpad = EPAD - E
    dstp = jnp.concatenate(
        [g[1], jnp.asarray(N + (np.arange(dpad) % (N_PAD - N)),
                           dtype=jnp.int32)])
    dstp = dstp.reshape(EPAD // CH, CH)

    mpad = MEPAD - E
    pidx = np.arange(mpad)
    msrc = jnp.concatenate(
        [g[0], jnp.asarray(N + (pidx % _ZPAD), dtype=jnp.int32)])
    mdst = jnp.concatenate(
        [g[1], jnp.asarray(pidx % N, dtype=jnp.int32)])
    gidx = jnp.stack([msrc.reshape(MEPAD // MCH, MCH),
                      mdst.reshape(MEPAD // MCH, MCH)], axis=1)

    degp = _sc_deg(dstp)
    hw0 = _tc_matmul(x, W0)
    s0, dn = _tc_scale(degp[0, :N], degp[1, :N], hw0)

    acc = _sc_msg(_pad_s(s0), gidx)
    s1 = _tc_layer(acc[0], acc[1], s0, dn, b0.reshape(1, D), W1)
    acc = _sc_msg(_pad_s(s1), gidx)
    s2 = _tc_layer(acc[0], acc[1], s1, dn, b1.reshape(1, D), W2)
    acc = _sc_msg(_pad_s(s2), gidx)
    return _tc_final(acc[0], acc[1], s2, dn, b2.reshape(1, D))
